# baseline (speedup 1.0000x reference)
import functools
import math

import jax
import jax.numpy as jnp
from jax import lax
from jax.experimental import pallas as pl
from jax.experimental.pallas import tpu as pltpu
from jax.experimental.pallas import tpu_sc as plsc

S = 64.0
MARGIN = 0.5
COS_M = math.cos(MARGIN)
SIN_M = math.sin(MARGIN)
CLIP = 0.999999

_RPW = 32
_RPC = 512


def _margin_vec(x):
    t = jnp.clip(x, -CLIP, CLIP)
    a = 1.0 - t * t
    c = math.sqrt(2.0)
    y = jnp.full((16,), jnp.float32(c))
    for k in range(1, 11):
        y = jnp.where(
            a < jnp.float32(4.0 ** -k),
            jnp.full((16,), jnp.float32(c * 2.0**k)),
            y,
        )
    for _ in range(5):
        y = y * (1.5 - 0.5 * a * y * y)
    return (S * COS_M) * t - (S * SIN_M) * (a * y)


def _sc_apply_body(cos_hbm, lab_hbm, out_hbm, lab_v, win_v, out_v, sem, osem):
    cid = lax.axis_index("c")
    sid = lax.axis_index("s")
    base = cid * _RPC + sid * _RPW
    pltpu.sync_copy(lab_hbm, lab_v.at[pl.ds(0, 1024)])
    lane = lax.broadcasted_iota(jnp.int32, (16,), 0)

    def band(g, carry):
        band_row = pl.multiple_of(base + g * 8, 8)
        lab16 = lab_v[pl.ds(band_row, 16)]
        labs = [lab16[i] for i in range(8)]
        cs = [pl.multiple_of((l >> 7) << 7, 128) for l in labs]
        ps = [l & 127 for l in labs]
        in_copies = [
            pltpu.async_copy(
                cos_hbm.at[pl.ds(band_row, 8), pl.ds(cs[j], 128)],
                win_v.at[j], sem,
            )
            for j in range(8)
        ]
        for cp in in_copies:
            cp.wait()
        out_copies = []
        for j in range(8):
            for i in range(8):
                for k in range(8):
                    x = win_v[j, i, pl.ds(k * 16, 16)]
                    out_v[j, i, pl.ds(k * 16, 16)] = jnp.clip(x, -CLIP, CLIP) * S
            for i in range(8):
                hit = cs[i] == cs[j]
                o_i = pl.multiple_of((ps[i] >> 4) << 4, 16)
                q_eff = jnp.where(hit, ps[i] & 15, jnp.int32(-1))
                tv = win_v[j, i, pl.ds(o_i, 16)]
                dv = out_v[j, i, pl.ds(o_i, 16)]
                out_v[j, i, pl.ds(o_i, 16)] = jnp.where(
                    lane == q_eff, _margin_vec(tv), dv
                )
            out_copies.append(
                pltpu.async_copy(
                    out_v.at[j],
                    out_hbm.at[pl.ds(band_row, 8), pl.ds(cs[j], 128)],
                    osem,
                )
            )
        for cp in out_copies:
            cp.wait()
        return carry

    lax.fori_loop(0, _RPW // 8, band, 0)


def _sc_apply(cosine, labels, out_ref):
    mesh = plsc.VectorSubcoreMesh(core_axis_name="c", subcore_axis_name="s")
    f = functools.partial(
        pl.kernel,
        out_type=(),
        mesh=mesh,
        scratch_types=[
            pltpu.VMEM((1040,), jnp.int32),
            pltpu.VMEM((8, 8, 128), jnp.float32),
            pltpu.VMEM((8, 8, 128), jnp.float32),
            pltpu.SemaphoreType.DMA,
            pltpu.SemaphoreType.DMA,
        ],
    )(_sc_apply_body)
    f(cosine, labels, out_ref)


def _dense_body(x_ref, o_ref):
    o_ref[...] = jnp.clip(x_ref[...], -CLIP, CLIP) * S


def kernel(cosine, labels):
    B, N = cosine.shape
    BM = 512
    BN = 4096
    grid = (B // BM, pl.cdiv(N, BN))
    dense = pl.pallas_call(
        _dense_body,
        grid=grid,
        in_specs=[pl.BlockSpec((BM, BN), lambda i, j: (i, j))],
        out_specs=pl.BlockSpec((BM, BN), lambda i, j: (i, j)),
        out_shape=jax.ShapeDtypeStruct((B, N), jnp.float32),
    )(cosine)
    ref = jax.new_ref(dense)
    _sc_apply(cosine, labels, ref)
    return ref[...]


# speedup vs baseline: 1.0199x; 1.0199x over previous
import functools
import math

import jax
import jax.numpy as jnp
from jax import lax
from jax.experimental import pallas as pl
from jax.experimental.pallas import tpu as pltpu
from jax.experimental.pallas import tpu_sc as plsc

S = 64.0
MARGIN = 0.5
COS_M = math.cos(MARGIN)
SIN_M = math.sin(MARGIN)
CLIP = 0.999999

_RPW = 32
_RPC = 512


def _margin_vec(x):
    t = jnp.clip(x, -CLIP, CLIP)
    a = 1.0 - t * t
    c = math.sqrt(2.0)
    y = jnp.full((16,), jnp.float32(c))
    for k in range(1, 11):
        y = jnp.where(
            a < jnp.float32(4.0 ** -k),
            jnp.full((16,), jnp.float32(c * 2.0**k)),
            y,
        )
    for _ in range(5):
        y = y * (1.5 - 0.5 * a * y * y)
    return (S * COS_M) * t - (S * SIN_M) * (a * y)


def _sc_margin_body(cos_hbm, lab_hbm, out_hbm, lab_v, win_v, m_v, sem, osem):
    cid = lax.axis_index("c")
    sid = lax.axis_index("s")
    base = cid * _RPC + sid * _RPW
    pltpu.sync_copy(lab_hbm, lab_v.at[pl.ds(0, 1024)])

    def band(g, carry):
        band_row = pl.multiple_of(base + g * 8, 8)
        lab16 = lab_v[pl.ds(band_row, 16)]
        labs = [lab16[i] for i in range(8)]
        cs = [pl.multiple_of((l >> 7) << 7, 128) for l in labs]
        in_copies = [
            pltpu.async_copy(
                cos_hbm.at[pl.ds(band_row, 8), pl.ds(cs[j], 128)],
                win_v.at[j], sem,
            )
            for j in range(8)
        ]
        for cp in in_copies:
            cp.wait()
        for j in range(8):
            o_j = pl.multiple_of(((labs[j] & 127) >> 4) << 4, 16)
            tv = win_v[j, j, pl.ds(o_j, 16)]
            m_v[pl.ds(j * 16, 16)] = _margin_vec(tv)
        pltpu.async_copy(
            m_v, out_hbm.at[pl.ds(pl.multiple_of(band_row * 16, 128), 128)], osem
        ).wait()
        return carry

    lax.fori_loop(0, _RPW // 8, band, 0)


def _sc_margins(cosine, labels):
    B = labels.shape[0]
    mesh = plsc.VectorSubcoreMesh(core_axis_name="c", subcore_axis_name="s")
    f = functools.partial(
        pl.kernel,
        out_type=jax.ShapeDtypeStruct((B * 16,), jnp.float32),
        mesh=mesh,
        scratch_types=[
            pltpu.VMEM((1040,), jnp.int32),
            pltpu.VMEM((8, 8, 128), jnp.float32),
            pltpu.VMEM((128,), jnp.float32),
            pltpu.SemaphoreType.DMA,
            pltpu.SemaphoreType.DMA,
        ],
    )(_sc_margin_body)
    return f(cosine, labels)


def _dense_body(lab_ref, marg_ref, x_ref, o_ref):
    j = pl.program_id(1)
    bn = x_ref.shape[1]
    x = x_ref[...]
    xc = jnp.clip(x, -CLIP, CLIP)
    lab = lab_ref[0, 0, :]
    m16 = marg_ref[0, :, :]
    lane16 = jax.lax.broadcasted_iota(jnp.int32, m16.shape, 1)
    tgt_m = jnp.sum(
        jnp.where(lane16 == (lab[:, None] & 15), m16, 0.0), axis=1
    )
    col = jax.lax.broadcasted_iota(jnp.int32, x.shape, 1) + j * bn
    mask = col == lab[:, None]
    o_ref[...] = jnp.where(mask, tgt_m[:, None], xc * S)


def kernel(cosine, labels):
    B, N = cosine.shape
    margins = _sc_margins(cosine, labels).reshape(B, 16)
    BM = 256
    BN = 4096
    grid = (B // BM, pl.cdiv(N, BN))
    labels3 = labels.reshape(B // BM, 1, BM)
    margins3 = margins.reshape(B // BM, BM, 16)
    return pl.pallas_call(
        _dense_body,
        grid=grid,
        in_specs=[
            pl.BlockSpec((1, 1, BM), lambda i, j: (i, 0, 0)),
            pl.BlockSpec((1, BM, 16), lambda i, j: (i, 0, 0)),
            pl.BlockSpec((BM, BN), lambda i, j: (i, j)),
        ],
        out_specs=pl.BlockSpec((BM, BN), lambda i, j: (i, j)),
        out_shape=jax.ShapeDtypeStruct((B, N), jnp.float32),
    )(labels3, margins3, cosine)
